# baseline (speedup 1.0000x reference)
import functools

import jax
import jax.numpy as jnp
import numpy as np
from jax import lax
from jax.experimental import pallas as pl
from jax.experimental.pallas import tpu as pltpu
from jax.experimental.pallas import tpu_sc as plsc

_MAX_CNT = 100000
_N_EDGES = 160000
_N_NODES = 10000
_D = 256
_EPS = 1e-6


def _threefry2x32(k0, k1, x0, x1):
    def rotl(x, d):
        return ((x << np.uint32(d)) | (x >> np.uint32(32 - d))).astype(
            np.uint32)

    rot = [(13, 15, 26, 6), (17, 29, 16, 24)]
    ks = [np.uint32(k0), np.uint32(k1),
          np.uint32(k0) ^ np.uint32(k1) ^ np.uint32(0x1BD11BDA)]
    x0 = (x0 + ks[0]).astype(np.uint32)
    x1 = (x1 + ks[1]).astype(np.uint32)
    for i in range(5):
        for r in rot[i % 2]:
            x0 = (x0 + x1).astype(np.uint32)
            x1 = x0 ^ rotl(x1, r)
        x0 = (x0 + ks[(i + 1) % 3]).astype(np.uint32)
        x1 = (x1 + ks[(i + 2) % 3] + np.uint32(i + 1)).astype(np.uint32)
    return x0, x1


def _fixed_randint(seed, n, span):
    with np.errstate(over="ignore"):
        kb0, kb1 = _threefry2x32(np.uint32(0), np.uint32(seed),
                                 np.zeros(2, np.uint32),
                                 np.arange(2, dtype=np.uint32))
        zeros = np.zeros(n, np.uint32)
        iota = np.arange(n, dtype=np.uint32)
        h0, h1 = _threefry2x32(kb0[0], kb1[0], zeros, iota)
        l0, l1 = _threefry2x32(kb0[1], kb1[1], zeros, iota)
        hi, lo = h0 ^ h1, l0 ^ l1
        span = np.uint32(span)
        mult = np.uint32(65536) % span
        mult = np.uint32(mult * mult) % span
        return ((hi % span * mult + lo % span) % span).astype(np.int32)


_SEL = _fixed_randint(42, _MAX_CNT, _N_EDGES)
_UIDX_NP, _CNT_NP = np.unique(_SEL, return_counts=True)
_U = len(_UIDX_NP)

_NC = 2
_NS = 16
_W = _NC * _NS
_K = 48
_GRP = 16
_UNROLL = 6
_P = -(-(-(-_U // _W)) // (_K * _UNROLL)) * (_K * _UNROLL)
_UP = _W * _P
_NCHUNK = _P // _K

_UIDX_PAD = np.full((_UP + 2 * _K,), _UIDX_NP[-1], np.int32)
_UIDX_PAD[:_U] = _UIDX_NP
_CNT_PAD = np.zeros((_UP,), np.float32)
_CNT_PAD[:_U] = _CNT_NP.astype(np.float32)

_SBUF = 5840
_lo8c = np.empty((_W,), np.int64)
for _w in range(_W):
    _lo = (int(_UIDX_PAD[_w * _P]) & ~7)
    _lo8c[_w] = min(_lo, _N_EDGES - _SBUF)
_REL_PAD = np.empty((_UP + 2 * _K,), np.int32)
for _w in range(_W):
    _end = _UP + 2 * _K if _w == _W - 1 else (_w + 1) * _P
    _REL_PAD[_w * _P:_end] = _UIDX_PAD[_w * _P:_end] - _lo8c[_w]
assert _REL_PAD.min() >= 0 and _REL_PAD.max() < _SBUF


def _sc_body(node_hbm, edge_hbm, src_hbm, dst_hbm, uidx_hbm, rel_hbm,
             dot_hbm, naa_hbm, nbb_hbm,
             uidx_v, rel_v, sspan_v, dspan_v, sid_v, did_v, e_v, s_v, t_v,
             dot_v, naa_v, nbb_v,
             sem_e, sem_s, sem_t):
    wid = lax.axis_index("s") * _NC + lax.axis_index("c")
    base = wid * _P
    pltpu.sync_copy(uidx_hbm.at[pl.ds(base, _P + 2 * _K)], uidx_v)
    pltpu.sync_copy(rel_hbm.at[pl.ds(base, _P + 2 * _K)], rel_v)
    lo8 = pl.multiple_of((uidx_v[pl.ds(0, _GRP)] - rel_v[pl.ds(0, _GRP)])[0],
                         8)
    pltpu.sync_copy(src_hbm.at[pl.ds(lo8, _SBUF)], sspan_v)
    pltpu.sync_copy(dst_hbm.at[pl.ds(lo8, _SBUF)], dspan_v)

    def _idx(c):
        return uidx_v.at[pl.ds(c * _K, _K)]

    def _assemble_ids(c, b):
        for g in range(_K // _GRP):
            r16 = rel_v[pl.ds(c * _K + g * _GRP, _GRP)]
            sid_v[b, pl.ds(g * _GRP, _GRP)] = plsc.load_gather(
                sspan_v, [r16])
            did_v[b, pl.ds(g * _GRP, _GRP)] = plsc.load_gather(
                dspan_v, [r16])

    _H = _K // 2

    def _half(ref, b, h):
        return ref.at[b, pl.ds(h * _H, _H)]

    def _issue_e(c, b):
        for h in range(2):
            pltpu.async_copy(edge_hbm.at[uidx_v.at[pl.ds(c * _K + h * _H,
                                                         _H)]],
                             _half(e_v, b, h), sem_e.at[b, h])

    def _issue_st(b):
        for h in range(2):
            pltpu.async_copy(node_hbm.at[sid_v.at[b, pl.ds(h * _H, _H)]],
                             _half(s_v, b, h), sem_s.at[b, h])
            pltpu.async_copy(node_hbm.at[did_v.at[b, pl.ds(h * _H, _H)]],
                             _half(t_v, b, h), sem_t.at[b, h])

    iota16 = lax.iota(jnp.int32, _GRP)

    def _wait_all(c, be, bn):
        for h in range(2):
            pltpu.make_async_copy(
                edge_hbm.at[uidx_v.at[pl.ds(c * _K + h * _H, _H)]],
                _half(e_v, be, h), sem_e.at[be, h]).wait()
            pltpu.make_async_copy(node_hbm.at[sid_v.at[bn, pl.ds(h * _H,
                                                                 _H)]],
                                  _half(s_v, bn, h), sem_s.at[bn, h]).wait()
            pltpu.make_async_copy(node_hbm.at[did_v.at[bn, pl.ds(h * _H,
                                                                 _H)]],
                                  _half(t_v, bn, h), sem_t.at[bn, h]).wait()

    def _compute(c, be, bn):
        ev, sv, tv = e_v.at[be], s_v.at[bn], t_v.at[bn]
        _wait_all(c, be, bn)
        for g in range(_K // _GRP):
            rows = iota16 + (g * _GRP)

            def dim_step(d4, carry):
                acc = list(carry)
                for q in range(4):
                    col = (iota16 + (d4 * 4 + q)) & (_D - 1)
                    e16 = plsc.load_gather(ev, [rows, col])
                    s16 = plsc.load_gather(sv, [rows, col])
                    t16 = plsc.load_gather(tv, [rows, col])
                    a16 = s16 + e16
                    acc[q] = acc[q] + a16 * t16
                    acc[4 + q] = acc[4 + q] + a16 * a16
                    acc[8 + q] = acc[8 + q] + t16 * t16
                return tuple(acc)

            z = jnp.zeros((_GRP,), jnp.float32)
            acc = lax.fori_loop(0, _D // 4, dim_step, (z,) * 12)
            o = c * _K + g * _GRP
            dot_v[pl.ds(o, _GRP)] = (acc[0] + acc[1]) + (acc[2] + acc[3])
            naa_v[pl.ds(o, _GRP)] = (acc[4] + acc[5]) + (acc[6] + acc[7])
            nbb_v[pl.ds(o, _GRP)] = (acc[8] + acc[9]) + (acc[10] + acc[11])

    _assemble_ids(0, 0)
    _issue_st(0)
    _issue_e(0, 0)
    _issue_e(1, 1)

    def outer(o6, _):
        c0 = o6 * _UNROLL
        for j in range(_UNROLL):
            c = c0 + j
            _assemble_ids(c + 1, (j + 1) % 2)
            _issue_st((j + 1) % 2)
            _issue_e(c + 2, (j + 2) % 3)
            _compute(c, j % 3, j % 2)
        return ()

    lax.fori_loop(0, _NCHUNK // _UNROLL, outer, ())
    for h in range(2):
        pltpu.make_async_copy(node_hbm.at[sid_v.at[0, pl.ds(h * _H, _H)]],
                              _half(s_v, 0, h), sem_s.at[0, h]).wait()
        pltpu.make_async_copy(node_hbm.at[did_v.at[0, pl.ds(h * _H, _H)]],
                              _half(t_v, 0, h), sem_t.at[0, h]).wait()
        pltpu.make_async_copy(
            edge_hbm.at[uidx_v.at[pl.ds(_NCHUNK * _K + h * _H, _H)]],
            _half(e_v, 0, h), sem_e.at[0, h]).wait()
        pltpu.make_async_copy(
            edge_hbm.at[uidx_v.at[pl.ds((_NCHUNK + 1) * _K + h * _H, _H)]],
            _half(e_v, 1, h), sem_e.at[1, h]).wait()
    pltpu.sync_copy(dot_v, dot_hbm.at[pl.ds(base, _P)])
    pltpu.sync_copy(naa_v, naa_hbm.at[pl.ds(base, _P)])
    pltpu.sync_copy(nbb_v, nbb_hbm.at[pl.ds(base, _P)])


_sc_kernel = functools.partial(
    pl.kernel,
    out_type=(
        jax.ShapeDtypeStruct((_UP,), jnp.float32),
        jax.ShapeDtypeStruct((_UP,), jnp.float32),
        jax.ShapeDtypeStruct((_UP,), jnp.float32),
    ),
    mesh=plsc.VectorSubcoreMesh(core_axis_name="c", subcore_axis_name="s"),
    compiler_params=pltpu.CompilerParams(use_tc_tiling_on_sc=False,
                                         needs_layout_passes=False),
    scratch_types=[
        pltpu.VMEM((_P + 2 * _K,), jnp.int32),
        pltpu.VMEM((_P + 2 * _K,), jnp.int32),
        pltpu.VMEM((_SBUF,), jnp.int32),
        pltpu.VMEM((_SBUF,), jnp.int32),
        pltpu.VMEM((2, _K), jnp.int32),
        pltpu.VMEM((2, _K), jnp.int32),
        pltpu.VMEM((3, _K, _D), jnp.float32),
        pltpu.VMEM((2, _K, _D), jnp.float32),
        pltpu.VMEM((2, _K, _D), jnp.float32),
        pltpu.VMEM((_P,), jnp.float32),
        pltpu.VMEM((_P,), jnp.float32),
        pltpu.VMEM((_P,), jnp.float32),
        pltpu.SemaphoreType.DMA((3, 2)),
        pltpu.SemaphoreType.DMA((2, 2)),
        pltpu.SemaphoreType.DMA((2, 2)),
    ],
)(_sc_body)


_C = 512
_R = _UP // _C


def _finish_body(dot_ref, naa_ref, nbb_ref, cnt_ref, out_ref):
    dot = dot_ref[...]
    na = jnp.maximum(jnp.sqrt(naa_ref[...]), _EPS)
    nb = jnp.maximum(jnp.sqrt(nbb_ref[...]), _EPS)
    sim = dot / (na * nb)
    out_ref[...] = jnp.sum(cnt_ref[...] * (1.0 - sim)).reshape(1, 1)


_finish = pl.pallas_call(
    _finish_body,
    out_shape=jax.ShapeDtypeStruct((1, 1), jnp.float32),
)


def kernel(node_embed, edge_embed, node_scores, edge_idx, labels,
           mini_batch_id):
    uidx = jnp.asarray(_UIDX_PAD)
    rel = jnp.asarray(_REL_PAD)
    cnt = jnp.asarray(_CNT_PAD)
    src_all = edge_idx[0].astype(jnp.int32)
    dst_all = edge_idx[1].astype(jnp.int32)
    dot, naa, nbb = _sc_kernel(
        node_embed, edge_embed, src_all, dst_all, uidx, rel)
    out = _finish(dot.reshape(_R, _C), naa.reshape(_R, _C),
                  nbb.reshape(_R, _C), cnt.reshape(_R, _C))
    return out[0, 0]


# speedup vs baseline: 1.5513x; 1.5513x over previous
import functools

import jax
import jax.numpy as jnp
import numpy as np
from jax import lax
from jax.experimental import pallas as pl
from jax.experimental.pallas import tpu as pltpu
from jax.experimental.pallas import tpu_sc as plsc

_MAX_CNT = 100000
_N_EDGES = 160000
_N_NODES = 10000
_D = 256
_EPS = 1e-6


def _threefry2x32(k0, k1, x0, x1):
    def rotl(x, d):
        return ((x << np.uint32(d)) | (x >> np.uint32(32 - d))).astype(
            np.uint32)

    rot = [(13, 15, 26, 6), (17, 29, 16, 24)]
    ks = [np.uint32(k0), np.uint32(k1),
          np.uint32(k0) ^ np.uint32(k1) ^ np.uint32(0x1BD11BDA)]
    x0 = (x0 + ks[0]).astype(np.uint32)
    x1 = (x1 + ks[1]).astype(np.uint32)
    for i in range(5):
        for r in rot[i % 2]:
            x0 = (x0 + x1).astype(np.uint32)
            x1 = x0 ^ rotl(x1, r)
        x0 = (x0 + ks[(i + 1) % 3]).astype(np.uint32)
        x1 = (x1 + ks[(i + 2) % 3] + np.uint32(i + 1)).astype(np.uint32)
    return x0, x1


def _fixed_randint(seed, n, span):
    with np.errstate(over="ignore"):
        kb0, kb1 = _threefry2x32(np.uint32(0), np.uint32(seed),
                                 np.zeros(2, np.uint32),
                                 np.arange(2, dtype=np.uint32))
        zeros = np.zeros(n, np.uint32)
        iota = np.arange(n, dtype=np.uint32)
        h0, h1 = _threefry2x32(kb0[0], kb1[0], zeros, iota)
        l0, l1 = _threefry2x32(kb0[1], kb1[1], zeros, iota)
        hi, lo = h0 ^ h1, l0 ^ l1
        span = np.uint32(span)
        mult = np.uint32(65536) % span
        mult = np.uint32(mult * mult) % span
        return ((hi % span * mult + lo % span) % span).astype(np.int32)


_SEL = _fixed_randint(42, _MAX_CNT, _N_EDGES)
_UIDX_NP, _CNT_NP = np.unique(_SEL, return_counts=True)
_U = len(_UIDX_NP)

_NC = 2
_NS = 16
_W = _NC * _NS
_K = 48
_GRP = 16
_UNROLL = 6
_P = -(-(-(-_U // _W)) // (_K * _UNROLL)) * (_K * _UNROLL)
_UP = _W * _P
_NCHUNK = _P // _K

_UIDX_PAD = np.full((_UP + 2 * _K,), _UIDX_NP[-1], np.int32)
_UIDX_PAD[:_U] = _UIDX_NP
_CNT_PAD = np.zeros((_UP,), np.float32)
_CNT_PAD[:_U] = _CNT_NP.astype(np.float32)

_SBUF = 5840
_lo8c = np.empty((_W,), np.int64)
for _w in range(_W):
    _lo = (int(_UIDX_PAD[_w * _P]) & ~7)
    _lo8c[_w] = min(_lo, _N_EDGES - _SBUF)
_REL_PAD = np.empty((_UP + 2 * _K,), np.int32)
for _w in range(_W):
    _end = _UP + 2 * _K if _w == _W - 1 else (_w + 1) * _P
    _REL_PAD[_w * _P:_end] = _UIDX_PAD[_w * _P:_end] - _lo8c[_w]
assert _REL_PAD.min() >= 0 and _REL_PAD.max() < _SBUF


def _sc_body(node_hbm, edge_hbm, src_hbm, dst_hbm, uidx_hbm, rel_hbm,
             dot_hbm, naa_hbm, nbb_hbm,
             uidx_v, rel_v, sspan_v, dspan_v, sid_v, did_v, e_v, s_v, t_v,
             dot_v, naa_v, nbb_v,
             sem_e, sem_s, sem_t):
    wid = lax.axis_index("s") * _NC + lax.axis_index("c")
    base = wid * _P
    pltpu.sync_copy(uidx_hbm.at[pl.ds(base, _P + 2 * _K)], uidx_v)
    pltpu.sync_copy(rel_hbm.at[pl.ds(base, _P + 2 * _K)], rel_v)
    lo8 = pl.multiple_of((uidx_v[pl.ds(0, _GRP)] - rel_v[pl.ds(0, _GRP)])[0],
                         8)
    pltpu.sync_copy(src_hbm.at[pl.ds(lo8, _SBUF)], sspan_v)
    pltpu.sync_copy(dst_hbm.at[pl.ds(lo8, _SBUF)], dspan_v)

    def _idx(c):
        return uidx_v.at[pl.ds(c * _K, _K)]

    def _assemble_ids(c, b):
        for g in range(_K // _GRP):
            r16 = rel_v[pl.ds(c * _K + g * _GRP, _GRP)]
            sid_v[b, pl.ds(g * _GRP, _GRP)] = plsc.load_gather(
                sspan_v, [r16])
            did_v[b, pl.ds(g * _GRP, _GRP)] = plsc.load_gather(
                dspan_v, [r16])

    def _issue_e(c, b):
        pltpu.async_copy(edge_hbm.at[_idx(c)], e_v.at[b], sem_e.at[b])

    def _issue_st(b):
        pltpu.async_copy(node_hbm.at[sid_v.at[b]], s_v.at[b], sem_s.at[b])
        pltpu.async_copy(node_hbm.at[did_v.at[b]], t_v.at[b], sem_t.at[b])

    iota16 = lax.iota(jnp.int32, _GRP)

    def _compute(c, be, bn):
        ev, sv, tv = e_v.at[be], s_v.at[bn], t_v.at[bn]
        pltpu.make_async_copy(edge_hbm.at[_idx(c)], ev, sem_e.at[be]).wait()
        pltpu.make_async_copy(node_hbm.at[sid_v.at[bn]], sv,
                              sem_s.at[bn]).wait()
        pltpu.make_async_copy(node_hbm.at[did_v.at[bn]], tv,
                              sem_t.at[bn]).wait()
        for g in range(_K // _GRP):
            rows = iota16 + (g * _GRP)

            def dim_step(d4, carry):
                acc = list(carry)
                for q in range(4):
                    col = (iota16 + (d4 * 4 + q)) & (_D - 1)
                    e16 = plsc.load_gather(ev, [rows, col])
                    s16 = plsc.load_gather(sv, [rows, col])
                    t16 = plsc.load_gather(tv, [rows, col])
                    a16 = s16 + e16
                    acc[q] = acc[q] + a16 * t16
                    acc[4 + q] = acc[4 + q] + a16 * a16
                    acc[8 + q] = acc[8 + q] + t16 * t16
                return tuple(acc)

            z = jnp.zeros((_GRP,), jnp.float32)
            acc = lax.fori_loop(0, _D // 4, dim_step, (z,) * 12)
            o = c * _K + g * _GRP
            dot_v[pl.ds(o, _GRP)] = (acc[0] + acc[1]) + (acc[2] + acc[3])
            naa_v[pl.ds(o, _GRP)] = (acc[4] + acc[5]) + (acc[6] + acc[7])
            nbb_v[pl.ds(o, _GRP)] = (acc[8] + acc[9]) + (acc[10] + acc[11])

    _assemble_ids(0, 0)
    _issue_st(0)
    _issue_e(0, 0)
    _issue_e(1, 1)

    def outer(o6, _):
        c0 = o6 * _UNROLL
        for j in range(_UNROLL):
            c = c0 + j
            _assemble_ids(c + 1, (j + 1) % 2)
            _issue_st((j + 1) % 2)
            _issue_e(c + 2, (j + 2) % 3)
            _compute(c, j % 3, j % 2)
        return ()

    lax.fori_loop(0, _NCHUNK // _UNROLL, outer, ())
    pltpu.make_async_copy(node_hbm.at[sid_v.at[0]], s_v.at[0],
                          sem_s.at[0]).wait()
    pltpu.make_async_copy(node_hbm.at[did_v.at[0]], t_v.at[0],
                          sem_t.at[0]).wait()
    pltpu.make_async_copy(edge_hbm.at[_idx(_NCHUNK)], e_v.at[0],
                          sem_e.at[0]).wait()
    pltpu.make_async_copy(edge_hbm.at[_idx(_NCHUNK + 1)], e_v.at[1],
                          sem_e.at[1]).wait()
    pltpu.sync_copy(dot_v, dot_hbm.at[pl.ds(base, _P)])
    pltpu.sync_copy(naa_v, naa_hbm.at[pl.ds(base, _P)])
    pltpu.sync_copy(nbb_v, nbb_hbm.at[pl.ds(base, _P)])


_sc_kernel = functools.partial(
    pl.kernel,
    out_type=(
        jax.ShapeDtypeStruct((_UP,), jnp.float32),
        jax.ShapeDtypeStruct((_UP,), jnp.float32),
        jax.ShapeDtypeStruct((_UP,), jnp.float32),
    ),
    mesh=plsc.VectorSubcoreMesh(core_axis_name="c", subcore_axis_name="s"),
    compiler_params=pltpu.CompilerParams(use_tc_tiling_on_sc=True,
                                         needs_layout_passes=False),
    scratch_types=[
        pltpu.VMEM((_P + 2 * _K,), jnp.int32),
        pltpu.VMEM((_P + 2 * _K,), jnp.int32),
        pltpu.VMEM((_SBUF,), jnp.int32),
        pltpu.VMEM((_SBUF,), jnp.int32),
        pltpu.VMEM((2, _K), jnp.int32),
        pltpu.VMEM((2, _K), jnp.int32),
        pltpu.VMEM((3, _K, _D), jnp.float32),
        pltpu.VMEM((2, _K, _D), jnp.float32),
        pltpu.VMEM((2, _K, _D), jnp.float32),
        pltpu.VMEM((_P,), jnp.float32),
        pltpu.VMEM((_P,), jnp.float32),
        pltpu.VMEM((_P,), jnp.float32),
        pltpu.SemaphoreType.DMA((3,)),
        pltpu.SemaphoreType.DMA((2,)),
        pltpu.SemaphoreType.DMA((2,)),
    ],
)(_sc_body)


_C = 512
_R = _UP // _C


def _finish_body(dot_ref, naa_ref, nbb_ref, cnt_ref, out_ref):
    dot = dot_ref[...]
    na = jnp.maximum(jnp.sqrt(naa_ref[...]), _EPS)
    nb = jnp.maximum(jnp.sqrt(nbb_ref[...]), _EPS)
    sim = dot / (na * nb)
    out_ref[...] = jnp.sum(cnt_ref[...] * (1.0 - sim)).reshape(1, 1)


_finish = pl.pallas_call(
    _finish_body,
    out_shape=jax.ShapeDtypeStruct((1, 1), jnp.float32),
)


def kernel(node_embed, edge_embed, node_scores, edge_idx, labels,
           mini_batch_id):
    uidx = jnp.asarray(_UIDX_PAD)
    rel = jnp.asarray(_REL_PAD)
    cnt = jnp.asarray(_CNT_PAD)
    src_all = edge_idx[0].astype(jnp.int32)
    dst_all = edge_idx[1].astype(jnp.int32)
    dot, naa, nbb = _sc_kernel(
        node_embed, edge_embed, src_all, dst_all, uidx, rel)
    out = _finish(dot.reshape(_R, _C), naa.reshape(_R, _C),
                  nbb.reshape(_R, _C), cnt.reshape(_R, _C))
    return out[0, 0]
